# baseline (device time: 86100 ns/iter reference)
import jax
import jax.numpy as jnp
from jax import lax
from jax.experimental import pallas as pl
from jax.experimental.pallas import tpu as pltpu

N_DEV = 4


def _gelu(y):
    c = 0.7978845608028654
    return 0.5 * y * (1.0 + jnp.tanh(c * (y + 0.044715 * y * y * y)))


def kernel(x, w_mat):
    m, k_per = x.shape
    _, n = w_mat.shape
    chunk = m // N_DEV

    def body(x_ref, w_ref, out_ref, comm_ref, send_sems, recv_sems):
        my = lax.axis_index("i")
        left = lax.rem(my + N_DEV - 1, N_DEV)
        right = lax.rem(my + 1, N_DEV)

        barrier = pltpu.get_barrier_semaphore()
        for nbr in (left, right):
            pl.semaphore_signal(
                barrier, inc=1,
                device_id=(nbr,), device_id_type=pl.DeviceIdType.MESH,
            )
        pl.semaphore_wait(barrier, 2)

        out_ref[:, :] = jnp.dot(
            x_ref[:, :], w_ref[:, :], preferred_element_type=jnp.float32
        )

        for s in range(N_DEV - 1):
            c_send = lax.rem(my + N_DEV - s, N_DEV)
            c_recv = lax.rem(my + N_DEV - s - 1, N_DEV)
            rdma = pltpu.make_async_remote_copy(
                src_ref=out_ref.at[pl.ds(c_send * chunk, chunk), :],
                dst_ref=comm_ref.at[s],
                send_sem=send_sems.at[s],
                recv_sem=recv_sems.at[s],
                device_id=(right,),
                device_id_type=pl.DeviceIdType.MESH,
            )
            rdma.start()
            rdma.wait()
            row = pl.ds(c_recv * chunk, chunk)
            out_ref[row, :] = out_ref[row, :] + comm_ref[s]

        c_own = lax.rem(my + 1, N_DEV)
        row = pl.ds(c_own * chunk, chunk)
        out_ref[row, :] = _gelu(out_ref[row, :])

        for t in range(N_DEV - 1):
            c_send = lax.rem(my + 1 + N_DEV - t, N_DEV)
            sl = pl.ds(c_send * chunk, chunk)
            rdma = pltpu.make_async_remote_copy(
                src_ref=out_ref.at[sl, :],
                dst_ref=out_ref.at[sl, :],
                send_sem=send_sems.at[N_DEV - 1 + t],
                recv_sem=recv_sems.at[N_DEV - 1 + t],
                device_id=(right,),
                device_id_type=pl.DeviceIdType.MESH,
            )
            rdma.start()
            rdma.wait()

    n_hops = 2 * (N_DEV - 1)
    return pl.pallas_call(
        body,
        out_shape=jax.ShapeDtypeStruct((m, n), jnp.float32),
        in_specs=[
            pl.BlockSpec(memory_space=pltpu.VMEM),
            pl.BlockSpec(memory_space=pltpu.VMEM),
        ],
        out_specs=pl.BlockSpec(memory_space=pltpu.VMEM),
        scratch_shapes=[
            pltpu.VMEM((N_DEV - 1, chunk, n), jnp.float32),
            pltpu.SemaphoreType.DMA((n_hops,)),
            pltpu.SemaphoreType.DMA((n_hops,)),
        ],
        compiler_params=pltpu.CompilerParams(collective_id=0),
    )(x, w_mat)


# device time: 52003 ns/iter; 1.6557x vs baseline; 1.6557x over previous
import jax
import jax.numpy as jnp
from jax import lax
from jax.experimental import pallas as pl
from jax.experimental.pallas import tpu as pltpu

N_DEV = 4
CW, CCW = 0, 1


def _gelu(y):
    c = 0.7978845608028654
    return 0.5 * y * (1.0 + jnp.tanh(c * (y + 0.044715 * y * y * y)))


def kernel(x, w_mat):
    m, k_per = x.shape
    _, n = w_mat.shape
    chunk = m // N_DEV
    half = n // 2

    def body(x_ref, w_ref, out_ref, cw_ref, ccw_ref, send_sems, recv_sems):
        my = lax.axis_index("i")
        left = lax.rem(my + N_DEV - 1, N_DEV)
        right = lax.rem(my + 1, N_DEV)

        barrier = pltpu.get_barrier_semaphore()
        for nbr in (left, right):
            pl.semaphore_signal(
                barrier, inc=1,
                device_id=(nbr,), device_id_type=pl.DeviceIdType.MESH,
            )
        pl.semaphore_wait(barrier, 2)

        def gemm_chunk(c):
            rows = pl.ds(c * chunk, chunk)
            out_ref[rows, :] = jnp.dot(
                x_ref[rows, :], w_ref[:, :],
                preferred_element_type=jnp.float32,
            )

        cw_cols = pl.ds(0, half)
        ccw_cols = pl.ds(half, half)

        def make_rs(direction, s):
            if direction == CW:
                c_send = lax.rem(my + N_DEV - s, N_DEV)
                cols, dst, tgt = cw_cols, cw_ref, right
            else:
                c_send = lax.rem(my + s, N_DEV)
                cols, dst, tgt = ccw_cols, ccw_ref, left
            return pltpu.make_async_remote_copy(
                src_ref=out_ref.at[pl.ds(c_send * chunk, chunk), cols],
                dst_ref=dst.at[s],
                send_sem=send_sems.at[direction, s],
                recv_sem=recv_sems.at[direction, s],
                device_id=(tgt,),
                device_id_type=pl.DeviceIdType.MESH,
            )

        def make_ag(direction, t):
            if direction == CW:
                c = lax.rem(my + 1 + N_DEV - t, N_DEV)
                cols, tgt = cw_cols, right
            else:
                c = lax.rem(my + N_DEV - 1 + t, N_DEV)
                cols, tgt = ccw_cols, left
            sl = pl.ds(c * chunk, chunk)
            return pltpu.make_async_remote_copy(
                src_ref=out_ref.at[sl, cols],
                dst_ref=out_ref.at[sl, cols],
                send_sem=send_sems.at[direction, N_DEV - 1 + t],
                recv_sem=recv_sems.at[direction, N_DEV - 1 + t],
                device_id=(tgt,),
                device_id_type=pl.DeviceIdType.MESH,
            )

        rs = {d: [make_rs(d, s) for s in range(N_DEV - 1)] for d in (CW, CCW)}
        ag = {d: [make_ag(d, t) for t in range(N_DEV - 1)] for d in (CW, CCW)}

        gemm_chunk(my)
        rs[CW][0].start()
        rs[CCW][0].start()
        gemm_chunk(left)
        gemm_chunk(right)
        gemm_chunk(lax.rem(my + 2, N_DEV))

        for s in range(N_DEV - 1):
            rs[CW][s].wait_recv()
            c_recv = lax.rem(my + N_DEV - s - 1, N_DEV)
            rows = pl.ds(c_recv * chunk, chunk)
            out_ref[rows, cw_cols] = out_ref[rows, cw_cols] + cw_ref[s]
            if s < N_DEV - 2:
                rs[CW][s + 1].start()
            rs[CCW][s].wait_recv()
            c_recv = lax.rem(my + s + 1, N_DEV)
            rows = pl.ds(c_recv * chunk, chunk)
            out_ref[rows, ccw_cols] = out_ref[rows, ccw_cols] + ccw_ref[s]
            if s < N_DEV - 2:
                rs[CCW][s + 1].start()

        rows = pl.ds(lax.rem(my + 1, N_DEV) * chunk, chunk)
        out_ref[rows, cw_cols] = _gelu(out_ref[rows, cw_cols])
        ag[CW][0].start()
        rows = pl.ds(left * chunk, chunk)
        out_ref[rows, ccw_cols] = _gelu(out_ref[rows, ccw_cols])
        ag[CCW][0].start()

        for t in range(N_DEV - 1):
            ag[CW][t].wait_recv()
            if t < N_DEV - 2:
                ag[CW][t + 1].start()
            ag[CCW][t].wait_recv()
            if t < N_DEV - 2:
                ag[CCW][t + 1].start()

        for d in (CW, CCW):
            for r in rs[d] + ag[d]:
                r.wait_send()

    n_hops = 2 * (N_DEV - 1)
    return pl.pallas_call(
        body,
        out_shape=jax.ShapeDtypeStruct((m, n), jnp.float32),
        in_specs=[
            pl.BlockSpec(memory_space=pltpu.VMEM),
            pl.BlockSpec(memory_space=pltpu.VMEM),
        ],
        out_specs=pl.BlockSpec(memory_space=pltpu.VMEM),
        scratch_shapes=[
            pltpu.VMEM((N_DEV - 1, chunk, half), jnp.float32),
            pltpu.VMEM((N_DEV - 1, chunk, half), jnp.float32),
            pltpu.SemaphoreType.DMA((2, n_hops)),
            pltpu.SemaphoreType.DMA((2, n_hops)),
        ],
        compiler_params=pltpu.CompilerParams(collective_id=0),
    )(x, w_mat)


# device time: 43771 ns/iter; 1.9671x vs baseline; 1.1881x over previous
import jax
import jax.numpy as jnp
from jax import lax
from jax.experimental import pallas as pl
from jax.experimental.pallas import tpu as pltpu

N_DEV = 4
N_STREAMS = 4


def _gelu(y):
    c = 0.7978845608028654
    return 0.5 * y * (1.0 + jnp.tanh(c * (y + 0.044715 * y * y * y)))


def kernel(x, w_mat):
    m, k_per = x.shape
    _, n = w_mat.shape
    chunk = m // N_DEV
    scol = n // N_STREAMS

    def body(x_ref, w_ref, out_ref, comm_ref, send_sems, recv_sems):
        my = lax.axis_index("i")
        left = lax.rem(my + N_DEV - 1, N_DEV)
        right = lax.rem(my + 1, N_DEV)

        barrier = pltpu.get_barrier_semaphore()
        for nbr in (left, right):
            pl.semaphore_signal(
                barrier, inc=1,
                device_id=(nbr,), device_id_type=pl.DeviceIdType.MESH,
            )
        pl.semaphore_wait(barrier, 2)

        def gemm_chunk(c):
            rows = pl.ds(c * chunk, chunk)
            out_ref[rows, :] = jnp.dot(
                x_ref[rows, :], w_ref[:, :],
                preferred_element_type=jnp.float32,
            )

        def cols_of(k):
            return pl.ds(k * scol, scol)

        def is_cw(k):
            return k < N_STREAMS // 2

        def make_rs(k, s):
            if is_cw(k):
                c_send = lax.rem(my + N_DEV - s, N_DEV)
                tgt = right
            else:
                c_send = lax.rem(my + s, N_DEV)
                tgt = left
            return pltpu.make_async_remote_copy(
                src_ref=out_ref.at[pl.ds(c_send * chunk, chunk), cols_of(k)],
                dst_ref=comm_ref.at[k, s],
                send_sem=send_sems.at[k, s],
                recv_sem=recv_sems.at[k, s],
                device_id=(tgt,),
                device_id_type=pl.DeviceIdType.MESH,
            )

        def rs_recv_chunk(k, s):
            if is_cw(k):
                return lax.rem(my + N_DEV - s - 1, N_DEV)
            return lax.rem(my + s + 1, N_DEV)

        def make_ag(k, t):
            if is_cw(k):
                c = lax.rem(my + 1 + N_DEV - t, N_DEV)
                tgt = right
            else:
                c = lax.rem(my + N_DEV - 1 + t, N_DEV)
                tgt = left
            sl = pl.ds(c * chunk, chunk)
            return pltpu.make_async_remote_copy(
                src_ref=out_ref.at[sl, cols_of(k)],
                dst_ref=out_ref.at[sl, cols_of(k)],
                send_sem=send_sems.at[k, N_DEV - 1 + t],
                recv_sem=recv_sems.at[k, N_DEV - 1 + t],
                device_id=(tgt,),
                device_id_type=pl.DeviceIdType.MESH,
            )

        ORDER = (0, 2, 1, 3)
        rs = {k: [make_rs(k, s) for s in range(N_DEV - 1)] for k in range(N_STREAMS)}
        ag = {k: [make_ag(k, t) for t in range(N_DEV - 1)] for k in range(N_STREAMS)}

        def acc(k, s):
            rows = pl.ds(rs_recv_chunk(k, s) * chunk, chunk)
            c = cols_of(k)
            out_ref[rows, c] = out_ref[rows, c] + comm_ref[k, s]

        gemm_chunk(my)
        for k in ORDER:
            rs[k][0].start()
        gemm_chunk(left)
        gemm_chunk(right)
        gemm_chunk(lax.rem(my + 2, N_DEV))

        for s in range(N_DEV - 2):
            for k in ORDER:
                rs[k][s].wait_recv()
                acc(k, s)
                rs[k][s + 1].start()
        for k in ORDER:
            rs[k][N_DEV - 2].wait_recv()
            acc(k, N_DEV - 2)
            own = lax.rem(my + 1, N_DEV) if is_cw(k) else left
            rows = pl.ds(own * chunk, chunk)
            c = cols_of(k)
            out_ref[rows, c] = _gelu(out_ref[rows, c])
            ag[k][0].start()

        for t in range(N_DEV - 2):
            for k in ORDER:
                ag[k][t].wait_recv()
                ag[k][t + 1].start()
        for k in ORDER:
            ag[k][N_DEV - 2].wait_recv()

        for k in range(N_STREAMS):
            for r in rs[k] + ag[k]:
                r.wait_send()

    n_hops = 2 * (N_DEV - 1)
    return pl.pallas_call(
        body,
        out_shape=jax.ShapeDtypeStruct((m, n), jnp.float32),
        in_specs=[
            pl.BlockSpec(memory_space=pltpu.VMEM),
            pl.BlockSpec(memory_space=pltpu.VMEM),
        ],
        out_specs=pl.BlockSpec(memory_space=pltpu.VMEM),
        scratch_shapes=[
            pltpu.VMEM((N_STREAMS, N_DEV - 1, chunk, scol), jnp.float32),
            pltpu.SemaphoreType.DMA((N_STREAMS, n_hops)),
            pltpu.SemaphoreType.DMA((N_STREAMS, n_hops)),
        ],
        compiler_params=pltpu.CompilerParams(collective_id=0),
    )(x, w_mat)
